# Initial kernel scaffold; baseline (speedup 1.0000x reference)
import jax
import jax.numpy as jnp
from jax.experimental import pallas as pl


def kernel(x, topk_indices, topk_weights, W1, W2, device_indices_map, local_expert_indices_map):
    raise NotImplementedError("write your pallas kernel here")


# speedup vs baseline: 1.3896x; 1.3896x over previous
import functools

import jax
import jax.numpy as jnp
from jax.experimental import pallas as pl
from jax.experimental.pallas import tpu as pltpu

_E = 8
_K = 2
_TB = 512


def _ffn_body(topk_ref, w_ref, lmap_ref, x_ref, w1_ref, w2_ref, y_ref):
    e = pl.program_id(1)
    topk = topk_ref[...]
    wts = w_ref[...]
    loc = jnp.zeros_like(topk)
    for j in range(_E):
        loc = jnp.where(topk == j, lmap_ref[j], loc)
    col = jnp.sum(jnp.where(loc == e, wts, 0.0), axis=1)
    h = jnp.maximum(
        jax.lax.dot_general(x_ref[...], w1_ref[...], (((1,), (0,)), ((), ())),
                            preferred_element_type=jnp.float32), 0.0)
    o = jax.lax.dot_general(h, w2_ref[...], (((1,), (0,)), ((), ())),
                            preferred_element_type=jnp.float32)
    contrib = col[:, None] * o

    @pl.when(e == 0)
    def _():
        y_ref[...] = contrib

    @pl.when(e != 0)
    def _():
        y_ref[...] += contrib


def kernel(x, topk_indices, topk_weights, W1, W2, device_indices_map,
           local_expert_indices_map):
    T, D = x.shape
    F = W1.shape[-1]
    grid = (T // _TB, _E)
    return pl.pallas_call(
        _ffn_body,
        grid=grid,
        in_specs=[
            pl.BlockSpec((_TB, _K), lambda tb, e: (tb, 0)),
            pl.BlockSpec((_TB, _K), lambda tb, e: (tb, 0)),
            pl.BlockSpec(memory_space=pltpu.SMEM),
            pl.BlockSpec((_TB, D), lambda tb, e: (tb, 0)),
            pl.BlockSpec((None, D, F), lambda tb, e: (e, 0, 0)),
            pl.BlockSpec((None, F, D), lambda tb, e: (e, 0, 0)),
        ],
        out_specs=pl.BlockSpec((_TB, D), lambda tb, e: (tb, 0)),
        out_shape=jax.ShapeDtypeStruct((T, D), jnp.float32),
    )(topk_indices, topk_weights, local_expert_indices_map, x, W1, W2)
